# baseline (speedup 1.0000x reference)
import jax
import jax.numpy as jnp
from jax import lax
from jax.experimental import pallas as pl
from jax.experimental.pallas import tpu as pltpu
from jax.experimental.pallas import tpu_sc as plsc

_B = 16384
_H = 200
_NUM_WORKERS = 32
_ROWS_W = _B // _NUM_WORKERS
_CROWS = 64
_CHUNK = _CROWS * _H
_N_CHUNKS = _ROWS_W // _CROWS
_LANES = 16

_OFFS = list(range(0, _H - _LANES + 1, _LANES)) + [_H - _LANES]


def _flatten_rows(src2d, dst_flat):
    def _body(r, carry):
        row = src2d.at[r]
        for o in _OFFS:
            dst_flat[pl.ds(r * _H + o, _LANES)] = row[pl.ds(o, _LANES)]
        return carry

    lax.fori_loop(0, _CROWS, _body, 0)


def _copy_to_2d(src_flat, dst2d):
    def _body(r, carry):
        row = dst2d.at[r]
        for o in _OFFS:
            row[pl.ds(o, _LANES)] = src_flat[pl.ds(r * _H + o, _LANES)]
        return carry

    lax.fori_loop(0, _CROWS, _body, 0)


def _witness_body(ids_hbm, tab_hbm, out_hbm,
                  idx2d_v0, idx2d_v1, idxf_v0, idxf_v1,
                  rowsf_v0, rowsf_v1, rows2d_v0, rows2d_v1,
                  gsem0, gsem1, osem0, osem1):
    wid = lax.axis_index("s") * 2 + lax.axis_index("c")
    rbase = wid * _ROWS_W
    idx2d_v = (idx2d_v0, idx2d_v1)
    idxf_v = (idxf_v0, idxf_v1)
    rowsf_v = (rowsf_v0, rowsf_v1)
    rows2d_v = (rows2d_v0, rows2d_v1)
    gsem = (gsem0, gsem1)
    osem = (osem0, osem1)

    def _stage_and_fire(g, b):
        pltpu.sync_copy(ids_hbm.at[pl.ds(rbase + g * _CROWS, _CROWS)],
                        idx2d_v[b])
        _flatten_rows(idx2d_v[b], idxf_v[b])
        return pltpu.async_copy(tab_hbm.at[idxf_v[b]], rowsf_v[b], gsem[b])

    gat = {0: _stage_and_fire(0, 0)}
    out_cp = {}

    for g in range(_N_CHUNKS):
        b = g % 2
        if g + 1 < _N_CHUNKS:
            gat[g + 1] = _stage_and_fire(g + 1, 1 - b)
        gat[g].wait()
        if g - 2 >= 0:
            out_cp[g - 2].wait()
        _copy_to_2d(rowsf_v[b], rows2d_v[b])
        out_cp[g] = pltpu.async_copy(
            rows2d_v[b], out_hbm.at[pl.ds(rbase + g * _CROWS, _CROWS)],
            osem[b])
    out_cp[_N_CHUNKS - 2].wait()
    out_cp[_N_CHUNKS - 1].wait()


def kernel(input_ids, witness_ids, witness_weight):
    del input_ids
    tab = witness_weight.reshape(-1)
    mesh = plsc.VectorSubcoreMesh(core_axis_name="c", subcore_axis_name="s")
    out = pl.kernel(
        _witness_body,
        out_type=jax.ShapeDtypeStruct((_B, _H), jnp.float32),
        mesh=mesh,
        scratch_types=[
            pltpu.VMEM((_CROWS, _H), jnp.int32),
            pltpu.VMEM((_CROWS, _H), jnp.int32),
            pltpu.VMEM((_CHUNK,), jnp.int32),
            pltpu.VMEM((_CHUNK,), jnp.int32),
            pltpu.VMEM((_CHUNK,), jnp.float32),
            pltpu.VMEM((_CHUNK,), jnp.float32),
            pltpu.VMEM((_CROWS, _H), jnp.float32),
            pltpu.VMEM((_CROWS, _H), jnp.float32),
            pltpu.SemaphoreType.DMA,
            pltpu.SemaphoreType.DMA,
            pltpu.SemaphoreType.DMA,
            pltpu.SemaphoreType.DMA,
        ],
    )(witness_ids, tab)
    w = out.reshape(_B, _H, 1)
    return w - lax.stop_gradient(w)


# speedup vs baseline: 1.4750x; 1.4750x over previous
import jax
import jax.numpy as jnp
from jax import lax
from jax.experimental import pallas as pl
from jax.experimental.pallas import tpu as pltpu
from jax.experimental.pallas import tpu_sc as plsc

_B = 16384
_H = 200
_NUM_WORKERS = 32
_COLS_W = _B // _NUM_WORKERS
_BCOLS = 128
_N_BLOCKS = _COLS_W // _BCOLS
_SCOLS = 64
_CHUNK = _H * _SCOLS
_N_STAGES = 2 * _N_BLOCKS
_LANES = 16
_NSUB = _SCOLS // _LANES


def _flatten_half(src2d, h, dst_flat):
    def _body(l, carry):
        row = src2d.at[l]
        for j in range(_NSUB):
            dst_flat[pl.ds(l * _SCOLS + j * _LANES, _LANES)] = (
                row[pl.ds(h * _SCOLS + j * _LANES, _LANES)])
        return carry

    lax.fori_loop(0, _H, _body, 0)


def _subtract_half(src_flat, dst3d, h):
    def _body(l, carry):
        row = dst3d.at[l].at[0]
        for j in range(_NSUB):
            v = src_flat[pl.ds(l * _SCOLS + j * _LANES, _LANES)]
            row[pl.ds(h * _SCOLS + j * _LANES, _LANES)] = v - v
        return carry

    lax.fori_loop(0, _H, _body, 0)


def _witness_body(ids_hbm, tab_hbm, out_hbm,
                  idx2d_v, idxf_v0, idxf_v1, rowsf_v0, rowsf_v1,
                  rows2d_v0, rows2d_v1,
                  gsem0, gsem1, osem0, osem1):
    wid = lax.axis_index("s") * 2 + lax.axis_index("c")
    cbase = wid * _COLS_W
    idxf_v = (idxf_v0, idxf_v1)
    rowsf_v = (rowsf_v0, rowsf_v1)
    rows2d_v = (rows2d_v0, rows2d_v1)
    gsem = (gsem0, gsem1)
    osem = (osem0, osem1)

    def _stage_block(blk):
        pltpu.sync_copy(ids_hbm.at[:, pl.ds(cbase + blk * _BCOLS, _BCOLS)],
                        idx2d_v)

    tab_1d = tab_hbm.at[0]

    def _fire_stage(s):
        b = s % 2
        _flatten_half(idx2d_v, s % 2, idxf_v[b])
        return pltpu.async_copy(tab_1d.at[idxf_v[b]], rowsf_v[b], gsem[b])

    _stage_block(0)
    gat = {0: _fire_stage(0)}
    out_cp = {}

    for s in range(_N_STAGES):
        b = s % 2
        if s + 1 < _N_STAGES:
            if (s + 1) % 2 == 0:
                _stage_block((s + 1) // 2)
            gat[s + 1] = _fire_stage(s + 1)
        gat[s].wait()
        blk = s // 2
        if s % 2 == 0 and blk - 2 >= 0:
            out_cp[blk - 2].wait()
        _subtract_half(rowsf_v[b], rows2d_v[blk % 2], s % 2)
        if s % 2 == 1:
            out_cp[blk] = pltpu.async_copy(
                rows2d_v[blk % 2],
                out_hbm.at[:, :, pl.ds(cbase + blk * _BCOLS, _BCOLS)],
                osem[blk % 2])
    out_cp[_N_BLOCKS - 2].wait()
    out_cp[_N_BLOCKS - 1].wait()


def kernel(input_ids, witness_ids, witness_weight):
    del input_ids
    ids_t = witness_ids.T
    tab2 = witness_weight.T
    mesh = plsc.VectorSubcoreMesh(core_axis_name="c", subcore_axis_name="s")
    out_t = pl.kernel(
        _witness_body,
        out_type=jax.ShapeDtypeStruct((_H, 1, _B), jnp.float32),
        mesh=mesh,
        scratch_types=[
            pltpu.VMEM((_H, _BCOLS), jnp.int32),
            pltpu.VMEM((_CHUNK,), jnp.int32),
            pltpu.VMEM((_CHUNK,), jnp.int32),
            pltpu.VMEM((_CHUNK,), jnp.float32),
            pltpu.VMEM((_CHUNK,), jnp.float32),
            pltpu.VMEM((_H, 1, _BCOLS), jnp.float32),
            pltpu.VMEM((_H, 1, _BCOLS), jnp.float32),
            pltpu.SemaphoreType.DMA,
            pltpu.SemaphoreType.DMA,
            pltpu.SemaphoreType.DMA,
            pltpu.SemaphoreType.DMA,
        ],
    )(ids_t, tab2)
    return out_t.transpose(2, 0, 1)


# speedup vs baseline: 1.6779x; 1.1375x over previous
import jax
import jax.numpy as jnp
from jax import lax
from jax.experimental import pallas as pl
from jax.experimental.pallas import tpu as pltpu
from jax.experimental.pallas import tpu_sc as plsc

_B = 16384
_H = 200
_NUM_WORKERS = 32
_COLS_W = _B // _NUM_WORKERS
_BCOLS = 128
_N_BLOCKS = _COLS_W // _BCOLS
_LANES = 16
_NSUB = _BCOLS // _LANES


def _subtract_in_place(dst3d):
    def _body(l, carry):
        row = dst3d.at[l].at[0]
        for j in range(_NSUB):
            v = row[pl.ds(j * _LANES, _LANES)]
            row[pl.ds(j * _LANES, _LANES)] = v - v
        return carry

    lax.fori_loop(0, _H, _body, 0)


def _witness_body(ids_hbm, tab_hbm, out_hbm,
                  idx2d_v0, idx2d_v1, rows3d_v0, rows3d_v1,
                  gsem0, gsem1, osem0, osem1):
    wid = lax.axis_index("s") * 2 + lax.axis_index("c")
    cbase = wid * _COLS_W
    idx2d_v = (idx2d_v0, idx2d_v1)
    rows3d_v = (rows3d_v0, rows3d_v1)
    gsem = (gsem0, gsem1)
    osem = (osem0, osem1)
    tab_1d = tab_hbm.at[0]

    def _out_slice(blk):
        return out_hbm.at[:, :, pl.ds(cbase + blk * _BCOLS, _BCOLS)]

    def _stage_and_fire(blk):
        b = blk % 2
        pltpu.sync_copy(ids_hbm.at[:, pl.ds(cbase + blk * _BCOLS, _BCOLS)],
                        idx2d_v[b])

        def _fire(l, carry):
            pltpu.async_copy(tab_1d.at[idx2d_v[b].at[l]],
                             rows3d_v[b].at[l].at[0], gsem[b])
            return carry

        lax.fori_loop(0, _H, _fire, 0)

    def _drain_gathers(blk):
        b = blk % 2
        pltpu.make_async_copy(_out_slice(blk), rows3d_v[b], gsem[b]).wait()

    _stage_and_fire(0)
    out_cp = {}

    for blk in range(_N_BLOCKS):
        b = blk % 2
        if blk + 1 < _N_BLOCKS:
            if blk - 1 >= 0:
                out_cp[blk - 1].wait()
            _stage_and_fire(blk + 1)
        _drain_gathers(blk)
        _subtract_in_place(rows3d_v[b])
        out_cp[blk] = pltpu.async_copy(rows3d_v[b], _out_slice(blk), osem[b])
    out_cp[_N_BLOCKS - 2].wait()
    out_cp[_N_BLOCKS - 1].wait()


def kernel(input_ids, witness_ids, witness_weight):
    del input_ids
    ids_t = witness_ids.T
    tab2 = witness_weight.T
    mesh = plsc.VectorSubcoreMesh(core_axis_name="c", subcore_axis_name="s")
    out_t = pl.kernel(
        _witness_body,
        out_type=jax.ShapeDtypeStruct((_H, 1, _B), jnp.float32),
        mesh=mesh,
        scratch_types=[
            pltpu.VMEM((_H, _BCOLS), jnp.int32),
            pltpu.VMEM((_H, _BCOLS), jnp.int32),
            pltpu.VMEM((_H, 1, _BCOLS), jnp.float32),
            pltpu.VMEM((_H, 1, _BCOLS), jnp.float32),
            pltpu.SemaphoreType.DMA,
            pltpu.SemaphoreType.DMA,
            pltpu.SemaphoreType.DMA,
            pltpu.SemaphoreType.DMA,
        ],
    )(ids_t, tab2)
    return out_t.transpose(2, 0, 1)
